# Initial kernel scaffold; baseline (speedup 1.0000x reference)
import functools

import jax
import jax.numpy as jnp
from jax import lax
from jax.experimental import pallas as pl
from jax.experimental.pallas import tpu as pltpu
from jax.experimental.pallas import tpu_sc as plsc


_RB = 512
_CB = 2048
_PREC = lax.Precision.HIGHEST


def _argmin_body(z_ref, c_ref, idx_ref, bv_ref, bi_ref):
    c = pl.program_id(1)
    zb = z_ref[...]
    cb = c_ref[...]
    s = lax.dot_general(
        zb, cb,
        dimension_numbers=(((1,), (1,)), ((), ())),
        precision=_PREC,
        preferred_element_type=jnp.float32,
    )
    x2 = jnp.sum(zb * zb, axis=1, keepdims=True)
    y2 = jnp.sum(cb * cb, axis=1)
    d = (x2 + y2[None, :]) - 2.0 * s

    m = jnp.min(d, axis=1, keepdims=True)
    lane = lax.broadcasted_iota(jnp.int32, d.shape, 1)
    li = jnp.min(jnp.where(d == m, lane, jnp.int32(2**30)),
                 axis=1, keepdims=True)
    gi = li + c * _CB

    @pl.when(c == 0)
    def _():
        bv_ref[...] = m
        bi_ref[...] = gi

    @pl.when(c > 0)
    def _():
        better = m < bv_ref[...]
        bi_ref[...] = jnp.where(better, gi, bi_ref[...])
        bv_ref[...] = jnp.where(better, m, bv_ref[...])

    @pl.when(c == pl.num_programs(1) - 1)
    def _():
        idx_ref[...] = bi_ref[...]


def _argmin_call(z, codebook, sub):
    n, _ = z.shape
    v, k = codebook.shape
    grid = (n // _RB, v // _CB)
    return pl.pallas_call(
        _argmin_body,
        grid=grid,
        in_specs=[
            pl.BlockSpec((_RB, k), lambda r, c: (r, sub)),
            pl.BlockSpec((_CB, k), lambda r, c: (c, 0)),
        ],
        out_specs=pl.BlockSpec((_RB, 1), lambda r, c: (r, 0)),
        out_shape=jax.ShapeDtypeStruct((n, 1), jnp.int32),
        scratch_shapes=[
            pltpu.VMEM((_RB, 1), jnp.float32),
            pltpu.VMEM((_RB, 1), jnp.int32),
        ],
        compiler_params=pltpu.CompilerParams(
            dimension_semantics=("arbitrary", "arbitrary"),
        ),
    )(z, codebook)


def _gather_call(table, gidx):
    b = gidx.shape[0]
    d = table.shape[1]
    info = plsc.get_sparse_core_info()
    nw = info.num_cores * info.num_subcores
    nc = info.num_cores
    b_per_w = b // nw
    mesh = plsc.VectorSubcoreMesh(core_axis_name="c", subcore_axis_name="s")

    @functools.partial(
        pl.kernel,
        mesh=mesh,
        out_type=jax.ShapeDtypeStruct((b, d), jnp.float32),
        scratch_types=[
            pltpu.VMEM((b_per_w,), jnp.int32),
            pltpu.VMEM((b_per_w, d), jnp.float32),
            pltpu.SemaphoreType.DMA,
        ],
    )
    def gather_kernel(table_hbm, idx_hbm, out_hbm, idx_v, rows_v, sem):
        wid = lax.axis_index("s") * nc + lax.axis_index("c")
        base = wid * b_per_w
        pltpu.sync_copy(idx_hbm.at[pl.ds(base, b_per_w)], idx_v)
        pltpu.async_copy(table_hbm.at[idx_v], rows_v, sem).wait()
        pltpu.sync_copy(rows_v, out_hbm.at[pl.ds(base, b_per_w)])

    return gather_kernel(table, gidx)


def kernel(embeddings, codebook0, codebook1):
    batch, seq, emb = embeddings.shape
    z = embeddings.reshape(-1, emb)
    v = codebook0.shape[0]

    idx0 = _argmin_call(z, codebook0, 0)
    idx1 = _argmin_call(z, codebook1, 1)
    all_idx = jnp.concatenate([idx0, idx1], axis=1)

    table = jnp.concatenate([codebook0, codebook1], axis=0)
    gidx = (all_idx + jnp.array([0, v], jnp.int32)[None, :]).reshape(-1)
    quantized = _gather_call(table, gidx)

    return (quantized.reshape(batch, seq, emb),
            all_idx.reshape(batch, seq, 2))


# speedup vs baseline: 1.5209x; 1.5209x over previous
import functools

import jax
import jax.numpy as jnp
from jax import lax
from jax.experimental import pallas as pl
from jax.experimental.pallas import tpu as pltpu
from jax.experimental.pallas import tpu_sc as plsc


_RB = 512
_CB = 2048


def _argmin_body(z_ref, c_ref, idx_ref, bv_ref, bi_ref):
    c = pl.program_id(1)
    zb = z_ref[...]
    cb = c_ref[...]
    s = lax.dot_general(
        zb.astype(jnp.bfloat16), cb.astype(jnp.bfloat16),
        dimension_numbers=(((1,), (1,)), ((), ())),
        preferred_element_type=jnp.float32,
    )
    x2 = jnp.sum(zb * zb, axis=1, keepdims=True)
    y2 = jnp.sum(cb * cb, axis=1)
    d = (x2 + y2[None, :]) - 2.0 * s

    m = jnp.min(d, axis=1, keepdims=True)
    lane = lax.broadcasted_iota(jnp.int32, d.shape, 1)
    li = jnp.min(jnp.where(d == m, lane, jnp.int32(2**30)),
                 axis=1, keepdims=True)
    gi = li + c * _CB

    @pl.when(c == 0)
    def _():
        bv_ref[...] = m
        bi_ref[...] = gi

    @pl.when(c > 0)
    def _():
        better = m < bv_ref[...]
        bi_ref[...] = jnp.where(better, gi, bi_ref[...])
        bv_ref[...] = jnp.where(better, m, bv_ref[...])

    @pl.when(c == pl.num_programs(1) - 1)
    def _():
        idx_ref[...] = bi_ref[...]


def _argmin_call(z, codebook, sub):
    n, _ = z.shape
    v, k = codebook.shape
    grid = (n // _RB, v // _CB)
    return pl.pallas_call(
        _argmin_body,
        grid=grid,
        in_specs=[
            pl.BlockSpec((_RB, k), lambda r, c: (r, sub)),
            pl.BlockSpec((_CB, k), lambda r, c: (c, 0)),
        ],
        out_specs=pl.BlockSpec((_RB, 1), lambda r, c: (r, 0)),
        out_shape=jax.ShapeDtypeStruct((n, 1), jnp.int32),
        scratch_shapes=[
            pltpu.VMEM((_RB, 1), jnp.float32),
            pltpu.VMEM((_RB, 1), jnp.int32),
        ],
        compiler_params=pltpu.CompilerParams(
            dimension_semantics=("arbitrary", "arbitrary"),
        ),
    )(z, codebook)


def _gather_call(table, gidx):
    b = gidx.shape[0]
    d = table.shape[1]
    info = plsc.get_sparse_core_info()
    nw = info.num_cores * info.num_subcores
    nc = info.num_cores
    b_per_w = b // nw
    mesh = plsc.VectorSubcoreMesh(core_axis_name="c", subcore_axis_name="s")

    @functools.partial(
        pl.kernel,
        mesh=mesh,
        out_type=jax.ShapeDtypeStruct((b, d), jnp.float32),
        scratch_types=[
            pltpu.VMEM((b_per_w,), jnp.int32),
            pltpu.VMEM((b_per_w, d), jnp.float32),
            pltpu.SemaphoreType.DMA,
        ],
    )
    def gather_kernel(table_hbm, idx_hbm, out_hbm, idx_v, rows_v, sem):
        wid = lax.axis_index("s") * nc + lax.axis_index("c")
        base = wid * b_per_w
        pltpu.sync_copy(idx_hbm.at[pl.ds(base, b_per_w)], idx_v)
        pltpu.async_copy(table_hbm.at[idx_v], rows_v, sem).wait()
        pltpu.sync_copy(rows_v, out_hbm.at[pl.ds(base, b_per_w)])

    return gather_kernel(table, gidx)


def kernel(embeddings, codebook0, codebook1):
    batch, seq, emb = embeddings.shape
    z = embeddings.reshape(-1, emb)
    v = codebook0.shape[0]

    idx0 = _argmin_call(z, codebook0, 0)
    idx1 = _argmin_call(z, codebook1, 1)
    all_idx = jnp.concatenate([idx0, idx1], axis=1)

    table = jnp.concatenate([codebook0, codebook1], axis=0)
    gidx = (all_idx + jnp.array([0, v], jnp.int32)[None, :]).reshape(-1)
    quantized = _gather_call(table, gidx)

    return (quantized.reshape(batch, seq, emb),
            all_idx.reshape(batch, seq, 2))


# speedup vs baseline: 2.0143x; 1.3244x over previous
import functools

import jax
import jax.numpy as jnp
from jax import lax
from jax.experimental import pallas as pl
from jax.experimental.pallas import tpu as pltpu
from jax.experimental.pallas import tpu_sc as plsc


_RB = 512
_CB = 2048


def _argmin_body(z_ref, c_ref, idx_ref, c16_ref, y2_ref):
    r = pl.program_id(0)
    v = c_ref.shape[0]

    @pl.when(r == 0)
    def _():
        cb = c_ref[...]
        c16_ref[...] = cb.astype(jnp.bfloat16)
        y2_ref[...] = jnp.sum(cb * cb, axis=1).reshape(1, v)

    zb32 = z_ref[...]
    x2 = jnp.sum(zb32 * zb32, axis=1, keepdims=True)
    zb = (-2.0 * zb32).astype(jnp.bfloat16)

    nlanes = 128
    ncols = _CB // nlanes
    bv = bi = None
    for k in range(v // _CB):
        cbk = c16_ref[pl.ds(k * _CB, _CB), :]
        s = lax.dot_general(
            zb, cbk,
            dimension_numbers=(((1,), (1,)), ((), ())),
            preferred_element_type=jnp.float32,
        )
        y2k = y2_ref[:, pl.ds(k * _CB, _CB)]
        d = (x2 + y2k) + s

        p = d[:, 0:nlanes]
        jcol = jnp.zeros(p.shape, jnp.int32)
        for j in range(1, ncols):
            dj = d[:, j * nlanes:(j + 1) * nlanes]
            lt = dj < p
            p = jnp.where(lt, dj, p)
            jcol = jnp.where(lt, j, jcol)
        m = jnp.min(p, axis=1, keepdims=True)
        lane = lax.broadcasted_iota(jnp.int32, p.shape, 1)
        key = jcol * nlanes + lane
        li = jnp.min(jnp.where(p == m, key, jnp.int32(2**30)),
                     axis=1, keepdims=True) + k * _CB
        if k == 0:
            bv, bi = m, li
        else:
            better = m < bv
            bi = jnp.where(better, li, bi)
            bv = jnp.where(better, m, bv)
    idx_ref[...] = bi


def _argmin_call(z, codebook, sub):
    n, _ = z.shape
    v, k = codebook.shape
    return pl.pallas_call(
        _argmin_body,
        grid=(n // _RB,),
        in_specs=[
            pl.BlockSpec((_RB, k), lambda r: (r, sub)),
            pl.BlockSpec((v, k), lambda r: (0, 0)),
        ],
        out_specs=pl.BlockSpec((_RB, 1), lambda r: (r, 0)),
        out_shape=jax.ShapeDtypeStruct((n, 1), jnp.int32),
        scratch_shapes=[
            pltpu.VMEM((v, k), jnp.bfloat16),
            pltpu.VMEM((1, v), jnp.float32),
        ],
        compiler_params=pltpu.CompilerParams(
            dimension_semantics=("arbitrary",),
        ),
    )(z, codebook)


def _gather_call(table, gidx):
    b = gidx.shape[0]
    d = table.shape[1]
    info = plsc.get_sparse_core_info()
    nw = info.num_cores * info.num_subcores
    nc = info.num_cores
    b_per_w = b // nw
    mesh = plsc.VectorSubcoreMesh(core_axis_name="c", subcore_axis_name="s")

    @functools.partial(
        pl.kernel,
        mesh=mesh,
        out_type=jax.ShapeDtypeStruct((b, d), jnp.float32),
        scratch_types=[
            pltpu.VMEM((b_per_w,), jnp.int32),
            pltpu.VMEM((b_per_w, d), jnp.float32),
            pltpu.SemaphoreType.DMA,
        ],
    )
    def gather_kernel(table_hbm, idx_hbm, out_hbm, idx_v, rows_v, sem):
        wid = lax.axis_index("s") * nc + lax.axis_index("c")
        base = wid * b_per_w
        pltpu.sync_copy(idx_hbm.at[pl.ds(base, b_per_w)], idx_v)
        pltpu.async_copy(table_hbm.at[idx_v], rows_v, sem).wait()
        pltpu.sync_copy(rows_v, out_hbm.at[pl.ds(base, b_per_w)])

    return gather_kernel(table, gidx)


def kernel(embeddings, codebook0, codebook1):
    batch, seq, emb = embeddings.shape
    z = embeddings.reshape(-1, emb)
    v = codebook0.shape[0]

    idx0 = _argmin_call(z, codebook0, 0)
    idx1 = _argmin_call(z, codebook1, 1)
    all_idx = jnp.concatenate([idx0, idx1], axis=1)

    table = jnp.concatenate([codebook0, codebook1], axis=0)
    gidx = (all_idx + jnp.array([0, v], jnp.int32)[None, :]).reshape(-1)
    quantized = _gather_call(table, gidx)

    return (quantized.reshape(batch, seq, emb),
            all_idx.reshape(batch, seq, 2))
